# baseline (speedup 1.0000x reference)
import dataclasses
import functools

import jax
import jax.numpy as jnp
from jax import lax
from jax.experimental import pallas as pl
from jax.experimental.pallas import tpu as pltpu
from jax.experimental.pallas import tpu_sc as plsc

_L = 16
_CH = 128
_D = 128


def _norm_body(x_ref, o_ref):
    x = x_ref[...]
    ss = jnp.sum(x * x, axis=1, keepdims=True)
    n = jnp.sqrt(ss)
    o_ref[...] = x / jnp.maximum(n, 1e-12)


def _normalize_rows(x):
    return pl.pallas_call(
        _norm_body,
        out_shape=jax.ShapeDtypeStruct(x.shape, x.dtype),
    )(x)


def _bce_body(n_valid, p_ref, t_ref, o_ref):
    p = p_ref[...]
    t = t_ref[...]
    term = jnp.maximum(p, 0.0) - p * t + jnp.log1p(jnp.exp(-jnp.abs(p)))
    rows, cols = p.shape
    idx = (lax.broadcasted_iota(jnp.int32, (rows, cols), 0) * cols
           + lax.broadcasted_iota(jnp.int32, (rows, cols), 1))
    term = jnp.where(idx < n_valid, term, 0.0)
    o_ref[...] = jnp.sum(term, axis=0, keepdims=True)


def _bce_sum(p2d, t2d, n_valid):
    part = pl.pallas_call(
        functools.partial(_bce_body, n_valid),
        out_shape=jax.ShapeDtypeStruct((1, p2d.shape[1]), jnp.float32),
    )(p2d, t2d)
    return jnp.sum(part)


def _make_edge_kernel(nw, ppi_chunks, dti_chunks):
    ppw = ppi_chunks * _CH
    dtw = dti_chunks * _CH
    mesh = plsc.VectorSubcoreMesh(core_axis_name="c", subcore_axis_name="s")
    info = plsc.get_sparse_core_info()
    nc = info.num_cores

    cp = pltpu.CompilerParams()
    if "needs_layout_passes" in pltpu.CompilerParams.__dataclass_fields__:
        cp = dataclasses.replace(cp, needs_layout_passes=False)

    @functools.partial(
        pl.kernel,
        mesh=mesh,
        compiler_params=cp,
        out_type=jax.ShapeDtypeStruct((nw, 2, _L), jnp.float32),
        scratch_types=[
            pltpu.VMEM((_CH,), jnp.int32),
            pltpu.VMEM((_CH,), jnp.int32),
            pltpu.VMEM((_CH, _D), jnp.float32),
            pltpu.VMEM((_CH, _D), jnp.float32),
            pltpu.VMEM((2, _L), jnp.float32),
            pltpu.SemaphoreType.DMA,
            pltpu.SemaphoreType.DMA,
        ],
    )
    def edge_kernel(gene_hbm, drug_hbm, ps_hbm, pd_hbm, ds_hbm, dd_hbm,
                    out_hbm, sidx, didx, srows, drows, ovec, sem_a, sem_b):
        wid = lax.axis_index("s") * nc + lax.axis_index("c")

        def chunk_sum(src_tbl, dst_tbl, sidx_hbm, didx_hbm, base, acc):
            pltpu.sync_copy(sidx_hbm.at[pl.ds(base, _CH)], sidx)
            pltpu.sync_copy(didx_hbm.at[pl.ds(base, _CH)], didx)
            ca = pltpu.async_copy(src_tbl.at[sidx], srows, sem_a)
            cb = pltpu.async_copy(dst_tbl.at[didx], drows, sem_b)
            ca.wait()
            cb.wait()

            def edge(e, acc):
                prod = srows[e, pl.ds(0, _L)] * drows[e, pl.ds(0, _L)]
                for k in range(1, _D // _L):
                    prod = prod + (srows[e, pl.ds(k * _L, _L)]
                                   * drows[e, pl.ds(k * _L, _L)])
                dt = jnp.sum(prod)
                r = dt - 1.0
                return acc + r * r

            return lax.fori_loop(0, _CH, edge, acc)

        def ppi_step(c, acc):
            return chunk_sum(gene_hbm, gene_hbm, ps_hbm, pd_hbm,
                             (c * nw + wid) * _CH, acc)

        acc_ppi = lax.fori_loop(0, ppi_chunks, ppi_step,
                                jnp.zeros((), jnp.float32))

        def dti_step(c, acc):
            return chunk_sum(drug_hbm, gene_hbm, ds_hbm, dd_hbm,
                             (c * nw + wid) * _CH, acc)

        acc_dti = lax.fori_loop(0, dti_chunks, dti_step,
                                jnp.zeros((), jnp.float32))

        lane = lax.iota(jnp.int32, _L)
        ovec[0, :] = jnp.where(lane == 0, acc_ppi, 0.0)
        ovec[1, :] = jnp.where(lane == 0, acc_dti, 0.0)
        pltpu.sync_copy(ovec, out_hbm.at[wid])

    return edge_kernel


def _pad_idx(idx, total, fill):
    pad = total - idx.shape[0]
    if pad == 0:
        return idx.astype(jnp.int32)
    return jnp.concatenate(
        [idx.astype(jnp.int32),
         jnp.full((pad,), fill, dtype=jnp.int32)])


def kernel(gene_x, drug_x, predicted_dti, known_dti, ppi_edge_index,
           dti_src, dti_dst):
    dti_weight = 1.0
    topology_weight = 0.1

    n_gene, d = gene_x.shape
    n_drug = drug_x.shape[0]
    e_ppi = ppi_edge_index.shape[1]
    e_dti = predicted_dti.shape[0]

    info = plsc.get_sparse_core_info()
    nw = info.num_cores * info.num_subcores

    drug_rows = ((n_drug + _CH - 1) // _CH) * _CH + _CH
    drug_pad = jnp.concatenate(
        [drug_x, jnp.zeros((drug_rows - n_drug, d), drug_x.dtype)])
    gene_n = _normalize_rows(gene_x)
    drug_n = _normalize_rows(drug_pad)

    cols = 128
    n_flat = ((e_dti + cols * 8 - 1) // (cols * 8)) * (cols * 8)
    p2d = jnp.pad(predicted_dti, (0, n_flat - e_dti)).reshape(-1, cols)
    t2d = jnp.pad(known_dti, (0, n_flat - e_dti)).reshape(-1, cols)
    bce_total = _bce_sum(p2d, t2d, e_dti)

    per_block = nw * _CH

    def _even_chunks(n):
        c = (n + per_block - 1) // per_block
        return c + (c % 2)

    ppi_chunks = _even_chunks(e_ppi)
    dti_chunks = _even_chunks(e_dti)
    ppi_total = ppi_chunks * per_block
    dti_total = dti_chunks * per_block
    dti_pad = dti_total - e_dti

    ps = _pad_idx(ppi_edge_index[0], ppi_total, 0)
    pd = _pad_idx(ppi_edge_index[1], ppi_total, 0)
    ds = _pad_idx(dti_src, dti_total, n_drug)
    dd = _pad_idx(dti_dst, dti_total, 0)

    edge_kernel = _make_edge_kernel(nw, ppi_chunks, dti_chunks)
    parts = edge_kernel(gene_n, drug_n, ps, pd, ds, dd)

    ppi_sum = jnp.sum(parts[:, 0, :])
    dti_sum = jnp.sum(parts[:, 1, :]) - jnp.float32(dti_pad)

    topology_loss = ppi_sum / e_ppi + dti_sum / e_dti
    dti_loss = bce_total / e_dti
    return dti_weight * dti_loss + topology_weight * topology_loss


# speedup vs baseline: 2.6392x; 2.6392x over previous
import dataclasses
import functools

import jax
import jax.numpy as jnp
from jax import lax
from jax.experimental import pallas as pl
from jax.experimental.pallas import tpu as pltpu
from jax.experimental.pallas import tpu_sc as plsc

_L = 16
_CH = 128
_D = 128


def _norm_body(x_ref, o_ref):
    x = x_ref[...]
    ss = jnp.sum(x * x, axis=1, keepdims=True)
    n = jnp.sqrt(ss)
    o_ref[...] = x / jnp.maximum(n, 1e-12)


def _normalize_rows(x):
    return pl.pallas_call(
        _norm_body,
        out_shape=jax.ShapeDtypeStruct(x.shape, x.dtype),
    )(x)


def _bce_body(n_valid, p_ref, t_ref, o_ref):
    p = p_ref[...]
    t = t_ref[...]
    term = jnp.maximum(p, 0.0) - p * t + jnp.log1p(jnp.exp(-jnp.abs(p)))
    rows, cols = p.shape
    idx = (lax.broadcasted_iota(jnp.int32, (rows, cols), 0) * cols
           + lax.broadcasted_iota(jnp.int32, (rows, cols), 1))
    term = jnp.where(idx < n_valid, term, 0.0)
    o_ref[...] = jnp.sum(term, axis=0, keepdims=True)


def _bce_sum(p2d, t2d, n_valid):
    part = pl.pallas_call(
        functools.partial(_bce_body, n_valid),
        out_shape=jax.ShapeDtypeStruct((1, p2d.shape[1]), jnp.float32),
    )(p2d, t2d)
    return jnp.sum(part)


def _make_edge_kernel(nw, ppi_chunks, dti_chunks):
    ppw = ppi_chunks * _CH
    dtw = dti_chunks * _CH
    mesh = plsc.VectorSubcoreMesh(core_axis_name="c", subcore_axis_name="s")
    info = plsc.get_sparse_core_info()
    nc = info.num_cores

    cp = pltpu.CompilerParams()
    if "needs_layout_passes" in pltpu.CompilerParams.__dataclass_fields__:
        cp = dataclasses.replace(cp, needs_layout_passes=False)

    @functools.partial(
        pl.kernel,
        mesh=mesh,
        compiler_params=cp,
        out_type=jax.ShapeDtypeStruct((nw, 2, _L), jnp.float32),
        scratch_types=[
            pltpu.VMEM((_CH,), jnp.int32),
            pltpu.VMEM((_CH,), jnp.int32),
            pltpu.VMEM((_CH, _D), jnp.float32),
            pltpu.VMEM((_CH, _D), jnp.float32),
            pltpu.VMEM((2, _L), jnp.float32),
            pltpu.SemaphoreType.DMA,
            pltpu.SemaphoreType.DMA,
        ],
    )
    def edge_kernel(gene_hbm, drug_hbm, ps_hbm, pd_hbm, ds_hbm, dd_hbm,
                    out_hbm, sidx, didx, srows, drows, ovec, sem_a, sem_b):
        wid = lax.axis_index("s") * nc + lax.axis_index("c")

        def chunk_sum(src_tbl, dst_tbl, sidx_hbm, didx_hbm, base, acc):
            pltpu.sync_copy(sidx_hbm.at[pl.ds(base, _CH)], sidx)
            pltpu.sync_copy(didx_hbm.at[pl.ds(base, _CH)], didx)
            ca = pltpu.async_copy(src_tbl.at[sidx], srows, sem_a)
            cb = pltpu.async_copy(dst_tbl.at[didx], drows, sem_b)
            ca.wait()
            cb.wait()

            def edge(e, acc):
                prod = srows[e, pl.ds(0, _L)] * drows[e, pl.ds(0, _L)]
                for k in range(1, _D // _L):
                    prod = prod + (srows[e, pl.ds(k * _L, _L)]
                                   * drows[e, pl.ds(k * _L, _L)])
                dt = jnp.sum(prod)
                r = dt - 1.0
                return acc + r * r

            return lax.fori_loop(0, _CH, edge, acc)

        def ppi_step(c, acc):
            return chunk_sum(gene_hbm, gene_hbm, ps_hbm, pd_hbm,
                             (c * nw + wid) * _CH, acc)

        acc_ppi = lax.fori_loop(0, ppi_chunks, ppi_step,
                                jnp.zeros((), jnp.float32))

        def dti_step(c, acc):
            return chunk_sum(drug_hbm, gene_hbm, ds_hbm, dd_hbm,
                             (c * nw + wid) * _CH, acc)

        acc_dti = lax.fori_loop(0, dti_chunks, dti_step,
                                jnp.zeros((), jnp.float32))

        lane = lax.iota(jnp.int32, _L)
        ovec[0, :] = jnp.where(lane == 0, acc_ppi, 0.0)
        ovec[1, :] = jnp.where(lane == 0, acc_dti, 0.0)
        pltpu.sync_copy(ovec, out_hbm.at[wid])

    return edge_kernel


def _pad_idx(idx, total, fill_base, fill_mod):
    pad = total - idx.shape[0]
    if pad == 0:
        return idx.astype(jnp.int32)
    fill = fill_base + (jnp.arange(pad, dtype=jnp.int32) % fill_mod)
    return jnp.concatenate([idx.astype(jnp.int32), fill])


def kernel(gene_x, drug_x, predicted_dti, known_dti, ppi_edge_index,
           dti_src, dti_dst):
    dti_weight = 1.0
    topology_weight = 0.1

    n_gene, d = gene_x.shape
    n_drug = drug_x.shape[0]
    e_ppi = ppi_edge_index.shape[1]
    e_dti = predicted_dti.shape[0]

    info = plsc.get_sparse_core_info()
    nw = info.num_cores * info.num_subcores

    drug_rows = ((n_drug + _CH - 1) // _CH) * _CH + _CH
    drug_pad = jnp.concatenate(
        [drug_x, jnp.zeros((drug_rows - n_drug, d), drug_x.dtype)])
    gene_n = _normalize_rows(gene_x)
    drug_n = _normalize_rows(drug_pad)

    cols = 128
    n_flat = ((e_dti + cols * 8 - 1) // (cols * 8)) * (cols * 8)
    p2d = jnp.pad(predicted_dti, (0, n_flat - e_dti)).reshape(-1, cols)
    t2d = jnp.pad(known_dti, (0, n_flat - e_dti)).reshape(-1, cols)
    bce_total = _bce_sum(p2d, t2d, e_dti)

    per_block = nw * _CH

    def _even_chunks(n):
        c = (n + per_block - 1) // per_block
        return c + (c % 2)

    ppi_chunks = _even_chunks(e_ppi)
    dti_chunks = _even_chunks(e_dti)
    ppi_total = ppi_chunks * per_block
    dti_total = dti_chunks * per_block
    dti_pad = dti_total - e_dti

    ps = _pad_idx(ppi_edge_index[0], ppi_total, 0, n_gene)
    pd = _pad_idx(ppi_edge_index[1], ppi_total, 0, n_gene)
    ds = _pad_idx(dti_src, dti_total, n_drug, _D)
    dd = _pad_idx(dti_dst, dti_total, 0, n_gene)

    edge_kernel = _make_edge_kernel(nw, ppi_chunks, dti_chunks)
    parts = edge_kernel(gene_n, drug_n, ps, pd, ds, dd)

    ppi_sum = jnp.sum(parts[:, 0, :])
    dti_sum = jnp.sum(parts[:, 1, :]) - jnp.float32(dti_pad)

    topology_loss = ppi_sum / e_ppi + dti_sum / e_dti
    dti_loss = bce_total / e_dti
    return dti_weight * dti_loss + topology_weight * topology_loss


# speedup vs baseline: 3.5710x; 1.3531x over previous
import dataclasses
import functools

import jax
import jax.numpy as jnp
from jax import lax
from jax.experimental import pallas as pl
from jax.experimental.pallas import tpu as pltpu
from jax.experimental.pallas import tpu_sc as plsc

_L = 16
_CH = 128
_D = 128


def _norm_body(x_ref, o_ref):
    x = x_ref[...]
    ss = jnp.sum(x * x, axis=1, keepdims=True)
    n = jnp.sqrt(ss)
    o_ref[...] = x / jnp.maximum(n, 1e-12)


def _normalize_rows(x):
    return pl.pallas_call(
        _norm_body,
        out_shape=jax.ShapeDtypeStruct(x.shape, x.dtype),
    )(x)


def _bce_body(n_valid, p_ref, t_ref, o_ref):
    p = p_ref[...]
    t = t_ref[...]
    term = jnp.maximum(p, 0.0) - p * t + jnp.log1p(jnp.exp(-jnp.abs(p)))
    rows, cols = p.shape
    idx = (lax.broadcasted_iota(jnp.int32, (rows, cols), 0) * cols
           + lax.broadcasted_iota(jnp.int32, (rows, cols), 1))
    term = jnp.where(idx < n_valid, term, 0.0)
    o_ref[...] = jnp.sum(term, axis=0, keepdims=True)


def _bce_sum(p2d, t2d, n_valid):
    part = pl.pallas_call(
        functools.partial(_bce_body, n_valid),
        out_shape=jax.ShapeDtypeStruct((1, p2d.shape[1]), jnp.float32),
    )(p2d, t2d)
    return jnp.sum(part)


def _make_edge_kernel(nw, ppi_chunks, dti_chunks):
    ppw = ppi_chunks * _CH
    dtw = dti_chunks * _CH
    mesh = plsc.VectorSubcoreMesh(core_axis_name="c", subcore_axis_name="s")
    info = plsc.get_sparse_core_info()
    nc = info.num_cores

    cp = pltpu.CompilerParams()
    if "needs_layout_passes" in pltpu.CompilerParams.__dataclass_fields__:
        cp = dataclasses.replace(cp, needs_layout_passes=False)

    @functools.partial(
        pl.kernel,
        mesh=mesh,
        compiler_params=cp,
        out_type=jax.ShapeDtypeStruct((nw, 2, _L), jnp.float32),
        scratch_types=[
            pltpu.VMEM((_CH,), jnp.int32),
            pltpu.VMEM((_CH,), jnp.int32),
            pltpu.VMEM((_CH,), jnp.int32),
            pltpu.VMEM((_CH,), jnp.int32),
            pltpu.VMEM((_CH, _D), jnp.float32),
            pltpu.VMEM((_CH, _D), jnp.float32),
            pltpu.VMEM((_CH, _D), jnp.float32),
            pltpu.VMEM((_CH, _D), jnp.float32),
            pltpu.VMEM((2, _L), jnp.float32),
            pltpu.SemaphoreType.DMA,
            pltpu.SemaphoreType.DMA,
            pltpu.SemaphoreType.DMA,
            pltpu.SemaphoreType.DMA,
        ],
    )
    def edge_kernel(gene_hbm, drug_hbm, ps_hbm, pd_hbm, ds_hbm, dd_hbm,
                    out_hbm, sidx_a, didx_a, sidx_b, didx_b,
                    srows_a, drows_a, srows_b, drows_b, ovec,
                    ss0, ss1, sd0, sd1):
        wid = lax.axis_index("s") * nc + lax.axis_index("c")

        def compute(sb, db, acc):
            def edge(e, acc):
                prod = sb[e, pl.ds(0, _L)] * db[e, pl.ds(0, _L)]
                for k in range(1, _D // _L):
                    prod = prod + (sb[e, pl.ds(k * _L, _L)]
                                   * db[e, pl.ds(k * _L, _L)])
                dt = jnp.sum(prod)
                r = dt - 1.0
                return acc + r * r

            return lax.fori_loop(0, _CH, edge, acc)

        def run_class(tbl_s, tbl_d, sidx_hbm, didx_hbm, nch, acc):
            def pair(i, acc):
                c0 = ((2 * i) * nw + wid) * _CH
                c1 = ((2 * i + 1) * nw + wid) * _CH
                pltpu.sync_copy(sidx_hbm.at[pl.ds(c0, _CH)], sidx_a)
                pltpu.sync_copy(didx_hbm.at[pl.ds(c0, _CH)], didx_a)
                da_s = pltpu.async_copy(tbl_s.at[sidx_a], srows_a, ss0)
                da_d = pltpu.async_copy(tbl_d.at[didx_a], drows_a, sd0)
                pltpu.sync_copy(sidx_hbm.at[pl.ds(c1, _CH)], sidx_b)
                pltpu.sync_copy(didx_hbm.at[pl.ds(c1, _CH)], didx_b)
                db_s = pltpu.async_copy(tbl_s.at[sidx_b], srows_b, ss1)
                db_d = pltpu.async_copy(tbl_d.at[didx_b], drows_b, sd1)
                da_s.wait()
                da_d.wait()
                acc = compute(srows_a, drows_a, acc)
                db_s.wait()
                db_d.wait()
                return compute(srows_b, drows_b, acc)

            return lax.fori_loop(0, nch // 2, pair, acc)

        acc_ppi = run_class(gene_hbm, gene_hbm, ps_hbm, pd_hbm, ppi_chunks,
                            jnp.zeros((), jnp.float32))
        acc_dti = run_class(drug_hbm, gene_hbm, ds_hbm, dd_hbm, dti_chunks,
                            jnp.zeros((), jnp.float32))

        lane = lax.iota(jnp.int32, _L)
        ovec[0, :] = jnp.where(lane == 0, acc_ppi, 0.0)
        ovec[1, :] = jnp.where(lane == 0, acc_dti, 0.0)
        pltpu.sync_copy(ovec, out_hbm.at[wid])

    return edge_kernel


def _pad_idx(idx, total, fill_base, fill_mod):
    pad = total - idx.shape[0]
    if pad == 0:
        return idx.astype(jnp.int32)
    fill = fill_base + (jnp.arange(pad, dtype=jnp.int32) % fill_mod)
    return jnp.concatenate([idx.astype(jnp.int32), fill])


def kernel(gene_x, drug_x, predicted_dti, known_dti, ppi_edge_index,
           dti_src, dti_dst):
    dti_weight = 1.0
    topology_weight = 0.1

    n_gene, d = gene_x.shape
    n_drug = drug_x.shape[0]
    e_ppi = ppi_edge_index.shape[1]
    e_dti = predicted_dti.shape[0]

    info = plsc.get_sparse_core_info()
    nw = info.num_cores * info.num_subcores

    drug_rows = ((n_drug + _CH - 1) // _CH) * _CH + _CH
    drug_pad = jnp.concatenate(
        [drug_x, jnp.zeros((drug_rows - n_drug, d), drug_x.dtype)])
    gene_n = _normalize_rows(gene_x)
    drug_n = _normalize_rows(drug_pad)

    cols = 128
    n_flat = ((e_dti + cols * 8 - 1) // (cols * 8)) * (cols * 8)
    p2d = jnp.pad(predicted_dti, (0, n_flat - e_dti)).reshape(-1, cols)
    t2d = jnp.pad(known_dti, (0, n_flat - e_dti)).reshape(-1, cols)
    bce_total = _bce_sum(p2d, t2d, e_dti)

    per_block = nw * _CH

    def _even_chunks(n):
        c = (n + per_block - 1) // per_block
        return c + (c % 2)

    ppi_chunks = _even_chunks(e_ppi)
    dti_chunks = _even_chunks(e_dti)
    ppi_total = ppi_chunks * per_block
    dti_total = dti_chunks * per_block
    dti_pad = dti_total - e_dti

    ps = _pad_idx(ppi_edge_index[0], ppi_total, 0, n_gene)
    pd = _pad_idx(ppi_edge_index[1], ppi_total, 0, n_gene)
    ds = _pad_idx(dti_src, dti_total, n_drug, _D)
    dd = _pad_idx(dti_dst, dti_total, 0, n_gene)

    edge_kernel = _make_edge_kernel(nw, ppi_chunks, dti_chunks)
    parts = edge_kernel(gene_n, drug_n, ps, pd, ds, dd)

    ppi_sum = jnp.sum(parts[:, 0, :])
    dti_sum = jnp.sum(parts[:, 1, :]) - jnp.float32(dti_pad)

    topology_loss = ppi_sum / e_ppi + dti_sum / e_dti
    dti_loss = bce_total / e_dti
    return dti_weight * dti_loss + topology_weight * topology_loss


# speedup vs baseline: 4.3095x; 1.2068x over previous
import dataclasses
import functools

import jax
import jax.numpy as jnp
from jax import lax
from jax.experimental import pallas as pl
from jax.experimental.pallas import tpu as pltpu
from jax.experimental.pallas import tpu_sc as plsc

_L = 16
_CH = 128
_D = 128


def _norm_body(x_ref, o_ref):
    x = x_ref[...]
    ss = jnp.sum(x * x, axis=1, keepdims=True)
    n = jnp.sqrt(ss)
    o_ref[...] = (x / jnp.maximum(n, 1e-12)).astype(o_ref.dtype)


def _normalize_rows(x, out_dtype):
    return pl.pallas_call(
        _norm_body,
        out_shape=jax.ShapeDtypeStruct(x.shape, out_dtype),
    )(x)


def _bce_body(n_valid, p_ref, t_ref, o_ref):
    p = p_ref[...]
    t = t_ref[...]
    term = jnp.maximum(p, 0.0) - p * t + jnp.log1p(jnp.exp(-jnp.abs(p)))
    rows, cols = p.shape
    idx = (lax.broadcasted_iota(jnp.int32, (rows, cols), 0) * cols
           + lax.broadcasted_iota(jnp.int32, (rows, cols), 1))
    term = jnp.where(idx < n_valid, term, 0.0)
    o_ref[...] = jnp.sum(term, axis=0, keepdims=True)


def _bce_sum(p2d, t2d, n_valid):
    part = pl.pallas_call(
        functools.partial(_bce_body, n_valid),
        out_shape=jax.ShapeDtypeStruct((1, p2d.shape[1]), jnp.float32),
    )(p2d, t2d)
    return jnp.sum(part)


def _make_edge_kernel(nw, ppi_chunks, dti_chunks):
    ppw = ppi_chunks * _CH
    dtw = dti_chunks * _CH
    mesh = plsc.VectorSubcoreMesh(core_axis_name="c", subcore_axis_name="s")
    info = plsc.get_sparse_core_info()
    nc = info.num_cores

    cp = pltpu.CompilerParams()
    if "needs_layout_passes" in pltpu.CompilerParams.__dataclass_fields__:
        cp = dataclasses.replace(cp, needs_layout_passes=False)
    if "use_tc_tiling_on_sc" in pltpu.CompilerParams.__dataclass_fields__:
        cp = dataclasses.replace(cp, use_tc_tiling_on_sc=False)

    @functools.partial(
        pl.kernel,
        mesh=mesh,
        compiler_params=cp,
        out_type=jax.ShapeDtypeStruct((nw, 2, _L), jnp.float32),
        scratch_types=[
            pltpu.VMEM((_CH,), jnp.int32),
            pltpu.VMEM((_CH,), jnp.int32),
            pltpu.VMEM((_CH,), jnp.int32),
            pltpu.VMEM((_CH,), jnp.int32),
            pltpu.VMEM((_CH, _D // 2), jnp.float32),
            pltpu.VMEM((_CH, _D // 2), jnp.float32),
            pltpu.VMEM((_CH, _D // 2), jnp.float32),
            pltpu.VMEM((_CH, _D // 2), jnp.float32),
            pltpu.VMEM((2, _L), jnp.float32),
            pltpu.SemaphoreType.DMA,
            pltpu.SemaphoreType.DMA,
            pltpu.SemaphoreType.DMA,
            pltpu.SemaphoreType.DMA,
        ],
    )
    def edge_kernel(gene_hbm, drug_hbm, ps_hbm, pd_hbm, ds_hbm, dd_hbm,
                    out_hbm, sidx_a, didx_a, sidx_b, didx_b,
                    srows_a, drows_a, srows_b, drows_b, ovec,
                    ss0, ss1, sd0, sd1):
        wid = lax.axis_index("s") * nc + lax.axis_index("c")

        def compute(sb, db, acc):
            def edge(e, acc):
                prod = None
                for k in range(_D // 2 // _L):
                    s = plsc.bitcast(sb[e, pl.ds(k * _L, _L)], jnp.bfloat16)
                    t = plsc.bitcast(db[e, pl.ds(k * _L, _L)], jnp.bfloat16)
                    st = s * t
                    prod = st if prod is None else prod + st
                lo, hi = plsc.unpack(prod, format=plsc.PackFormat.INTERLEAVED)
                dt = jnp.sum(lo + hi)
                r = dt - 1.0
                return acc + r * r

            return lax.fori_loop(0, _CH, edge, acc)

        def run_class(tbl_s, tbl_d, sidx_hbm, didx_hbm, nch, acc):
            def pair(i, acc):
                c0 = ((2 * i) * nw + wid) * _CH
                c1 = ((2 * i + 1) * nw + wid) * _CH
                pltpu.sync_copy(sidx_hbm.at[pl.ds(c0, _CH)], sidx_a)
                pltpu.sync_copy(didx_hbm.at[pl.ds(c0, _CH)], didx_a)
                da_s = pltpu.async_copy(tbl_s.at[sidx_a], srows_a, ss0)
                da_d = pltpu.async_copy(tbl_d.at[didx_a], drows_a, sd0)
                pltpu.sync_copy(sidx_hbm.at[pl.ds(c1, _CH)], sidx_b)
                pltpu.sync_copy(didx_hbm.at[pl.ds(c1, _CH)], didx_b)
                db_s = pltpu.async_copy(tbl_s.at[sidx_b], srows_b, ss1)
                db_d = pltpu.async_copy(tbl_d.at[didx_b], drows_b, sd1)
                da_s.wait()
                da_d.wait()
                acc = compute(srows_a, drows_a, acc)
                db_s.wait()
                db_d.wait()
                return compute(srows_b, drows_b, acc)

            return lax.fori_loop(0, nch // 2, pair, acc)

        acc_ppi = run_class(gene_hbm, gene_hbm, ps_hbm, pd_hbm, ppi_chunks,
                            jnp.zeros((), jnp.float32))
        acc_dti = run_class(drug_hbm, gene_hbm, ds_hbm, dd_hbm, dti_chunks,
                            jnp.zeros((), jnp.float32))

        lane = lax.iota(jnp.int32, _L)
        ovec[0, :] = jnp.where(lane == 0, acc_ppi, 0.0)
        ovec[1, :] = jnp.where(lane == 0, acc_dti, 0.0)
        pltpu.sync_copy(ovec, out_hbm.at[wid])

    return edge_kernel


def _pad_idx(idx, total, fill_base, fill_mod):
    pad = total - idx.shape[0]
    if pad == 0:
        return idx.astype(jnp.int32)
    fill = fill_base + (jnp.arange(pad, dtype=jnp.int32) % fill_mod)
    return jnp.concatenate([idx.astype(jnp.int32), fill])


def kernel(gene_x, drug_x, predicted_dti, known_dti, ppi_edge_index,
           dti_src, dti_dst):
    dti_weight = 1.0
    topology_weight = 0.1

    n_gene, d = gene_x.shape
    n_drug = drug_x.shape[0]
    e_ppi = ppi_edge_index.shape[1]
    e_dti = predicted_dti.shape[0]

    info = plsc.get_sparse_core_info()
    nw = info.num_cores * info.num_subcores

    drug_rows = ((n_drug + _CH - 1) // _CH) * _CH + _CH
    drug_pad = jnp.concatenate(
        [drug_x, jnp.zeros((drug_rows - n_drug, d), drug_x.dtype)])

    def _pack(nrm):
        return lax.bitcast_convert_type(
            nrm.reshape(nrm.shape[0], d // 2, 2), jnp.float32)

    gene_n = _pack(_normalize_rows(gene_x, jnp.bfloat16))
    drug_n = _pack(_normalize_rows(drug_pad, jnp.bfloat16))

    cols = 128
    n_flat = ((e_dti + cols * 8 - 1) // (cols * 8)) * (cols * 8)
    p2d = jnp.pad(predicted_dti, (0, n_flat - e_dti)).reshape(-1, cols)
    t2d = jnp.pad(known_dti, (0, n_flat - e_dti)).reshape(-1, cols)
    bce_total = _bce_sum(p2d, t2d, e_dti)

    per_block = nw * _CH

    def _even_chunks(n):
        c = (n + per_block - 1) // per_block
        return c + (c % 2)

    ppi_chunks = _even_chunks(e_ppi)
    dti_chunks = _even_chunks(e_dti)
    ppi_total = ppi_chunks * per_block
    dti_total = dti_chunks * per_block
    dti_pad = dti_total - e_dti

    ps = _pad_idx(ppi_edge_index[0], ppi_total, 0, n_gene)
    pd = _pad_idx(ppi_edge_index[1], ppi_total, 0, n_gene)
    ds = _pad_idx(dti_src, dti_total, n_drug, _D)
    dd = _pad_idx(dti_dst, dti_total, 0, n_gene)

    edge_kernel = _make_edge_kernel(nw, ppi_chunks, dti_chunks)
    parts = edge_kernel(gene_n, drug_n, ps, pd, ds, dd)

    ppi_sum = jnp.sum(parts[:, 0, :])
    dti_sum = jnp.sum(parts[:, 1, :]) - jnp.float32(dti_pad)

    topology_loss = ppi_sum / e_ppi + dti_sum / e_dti
    dti_loss = bce_total / e_dti
    return dti_weight * dti_loss + topology_weight * topology_loss


# speedup vs baseline: 6.9264x; 1.6072x over previous
import dataclasses
import functools

import jax
import jax.numpy as jnp
from jax import lax
from jax.experimental import pallas as pl
from jax.experimental.pallas import tpu as pltpu
from jax.experimental.pallas import tpu_sc as plsc

_L = 16
_CH = 128
_D = 128


def _norm_body(x_ref, o_ref):
    x = x_ref[...]
    ss = jnp.sum(x * x, axis=1, keepdims=True)
    n = jnp.sqrt(ss)
    o_ref[...] = (x / jnp.maximum(n, 1e-12)).astype(o_ref.dtype)


def _normalize_rows(x, out_dtype):
    return pl.pallas_call(
        _norm_body,
        out_shape=jax.ShapeDtypeStruct(x.shape, out_dtype),
    )(x)


def _bce_body(n_valid, p_ref, t_ref, o_ref):
    p = p_ref[...]
    t = t_ref[...]
    term = jnp.maximum(p, 0.0) - p * t + jnp.log1p(jnp.exp(-jnp.abs(p)))
    rows, cols = p.shape
    idx = (lax.broadcasted_iota(jnp.int32, (rows, cols), 0) * cols
           + lax.broadcasted_iota(jnp.int32, (rows, cols), 1))
    term = jnp.where(idx < n_valid, term, 0.0)
    o_ref[...] = jnp.sum(term, axis=0, keepdims=True)


def _bce_sum(p2d, t2d, n_valid):
    part = pl.pallas_call(
        functools.partial(_bce_body, n_valid),
        out_shape=jax.ShapeDtypeStruct((1, p2d.shape[1]), jnp.float32),
    )(p2d, t2d)
    return jnp.sum(part)


def _make_edge_kernel(nw, ppi_chunks, dti_chunks):
    ppw = ppi_chunks * _CH
    dtw = dti_chunks * _CH
    mesh = plsc.VectorSubcoreMesh(core_axis_name="c", subcore_axis_name="s")
    info = plsc.get_sparse_core_info()
    nc = info.num_cores

    cp = pltpu.CompilerParams()
    if "needs_layout_passes" in pltpu.CompilerParams.__dataclass_fields__:
        cp = dataclasses.replace(cp, needs_layout_passes=False)
    if "use_tc_tiling_on_sc" in pltpu.CompilerParams.__dataclass_fields__:
        cp = dataclasses.replace(cp, use_tc_tiling_on_sc=False)

    @functools.partial(
        pl.kernel,
        mesh=mesh,
        compiler_params=cp,
        out_type=jax.ShapeDtypeStruct((nw, 2, _L), jnp.float32),
        scratch_types=[
            pltpu.VMEM((ppw,), jnp.int32),
            pltpu.VMEM((ppw,), jnp.int32),
            pltpu.VMEM((dtw,), jnp.int32),
            pltpu.VMEM((dtw,), jnp.int32),
            pltpu.VMEM((_CH, _D // 2), jnp.float32),
            pltpu.VMEM((_CH, _D // 2), jnp.float32),
            pltpu.VMEM((_CH, _D // 2), jnp.float32),
            pltpu.VMEM((_CH, _D // 2), jnp.float32),
            pltpu.VMEM((2, _L), jnp.float32),
            pltpu.SemaphoreType.DMA,
            pltpu.SemaphoreType.DMA,
            pltpu.SemaphoreType.DMA,
            pltpu.SemaphoreType.DMA,
        ],
    )
    def edge_kernel(gene_hbm, drug_hbm, ps_hbm, pd_hbm, ds_hbm, dd_hbm,
                    out_hbm, psidx, pdidx, dsidx, ddidx,
                    srows_a, drows_a, srows_b, drows_b, ovec,
                    ss0, ss1, sd0, sd1):
        wid = lax.axis_index("s") * nc + lax.axis_index("c")

        pltpu.sync_copy(ps_hbm.at[pl.ds(wid * ppw, ppw)], psidx)
        pltpu.sync_copy(pd_hbm.at[pl.ds(wid * ppw, ppw)], pdidx)
        pltpu.sync_copy(ds_hbm.at[pl.ds(wid * dtw, dtw)], dsidx)
        pltpu.sync_copy(dd_hbm.at[pl.ds(wid * dtw, dtw)], ddidx)

        def compute(sb, db, acc):
            def edge(e, acc):
                prod = None
                for k in range(_D // 2 // _L):
                    s = plsc.bitcast(sb[e, pl.ds(k * _L, _L)], jnp.bfloat16)
                    t = plsc.bitcast(db[e, pl.ds(k * _L, _L)], jnp.bfloat16)
                    st = s * t
                    prod = st if prod is None else prod + st
                lo, hi = plsc.unpack(prod, format=plsc.PackFormat.INTERLEAVED)
                dt = jnp.sum(lo + hi)
                r = dt - 1.0
                return acc + r * r

            return lax.fori_loop(0, _CH, edge, acc)

        def run_class(tbl_s, tbl_d, sidx, didx, nch, acc):
            bufs = ((srows_a, drows_a, ss0, sd0),
                    (srows_b, drows_b, ss1, sd1))

            def start(t, b):
                sr, dr, sss, ssd = bufs[b]
                pltpu.async_copy(tbl_s.at[sidx.at[pl.ds(t * _CH, _CH)]],
                                 sr, sss)
                pltpu.async_copy(tbl_d.at[didx.at[pl.ds(t * _CH, _CH)]],
                                 dr, ssd)

            def wait(b):
                sr, dr, sss, ssd = bufs[b]
                pltpu.make_async_copy(tbl_s.at[sidx.at[pl.ds(0, _CH)]],
                                      sr, sss).wait()
                pltpu.make_async_copy(tbl_d.at[didx.at[pl.ds(0, _CH)]],
                                      dr, ssd).wait()

            start(0, 0)
            start(1, 1)

            def pair(i, acc):
                for b in (0, 1):
                    t = 2 * i + b
                    wait(b)

                    @pl.when(t + 2 < nch)
                    def _():
                        start(t + 2, b)

                    sr, dr = bufs[b][0], bufs[b][1]
                    acc = compute(sr, dr, acc)
                return acc

            return lax.fori_loop(0, nch // 2, pair, acc)

        acc_ppi = run_class(gene_hbm, gene_hbm, psidx, pdidx, ppi_chunks,
                            jnp.zeros((), jnp.float32))
        acc_dti = run_class(drug_hbm, gene_hbm, dsidx, ddidx, dti_chunks,
                            jnp.zeros((), jnp.float32))

        lane = lax.iota(jnp.int32, _L)
        ovec[0, :] = jnp.where(lane == 0, acc_ppi, 0.0)
        ovec[1, :] = jnp.where(lane == 0, acc_dti, 0.0)
        pltpu.sync_copy(ovec, out_hbm.at[wid])

    return edge_kernel


def _pad_idx(idx, total, fill_base, fill_mod):
    pad = total - idx.shape[0]
    if pad == 0:
        return idx.astype(jnp.int32)
    fill = fill_base + (jnp.arange(pad, dtype=jnp.int32) % fill_mod)
    return jnp.concatenate([idx.astype(jnp.int32), fill])


def kernel(gene_x, drug_x, predicted_dti, known_dti, ppi_edge_index,
           dti_src, dti_dst):
    dti_weight = 1.0
    topology_weight = 0.1

    n_gene, d = gene_x.shape
    n_drug = drug_x.shape[0]
    e_ppi = ppi_edge_index.shape[1]
    e_dti = predicted_dti.shape[0]

    info = plsc.get_sparse_core_info()
    nw = info.num_cores * info.num_subcores

    drug_rows = ((n_drug + _CH - 1) // _CH) * _CH + _CH
    drug_pad = jnp.concatenate(
        [drug_x, jnp.zeros((drug_rows - n_drug, d), drug_x.dtype)])

    def _pack(nrm):
        return lax.bitcast_convert_type(
            nrm.reshape(nrm.shape[0], d // 2, 2), jnp.float32)

    gene_n = _pack(_normalize_rows(gene_x, jnp.bfloat16))
    drug_n = _pack(_normalize_rows(drug_pad, jnp.bfloat16))

    cols = 128
    n_flat = ((e_dti + cols * 8 - 1) // (cols * 8)) * (cols * 8)
    p2d = jnp.pad(predicted_dti, (0, n_flat - e_dti)).reshape(-1, cols)
    t2d = jnp.pad(known_dti, (0, n_flat - e_dti)).reshape(-1, cols)
    bce_total = _bce_sum(p2d, t2d, e_dti)

    per_block = nw * _CH

    def _even_chunks(n):
        c = (n + per_block - 1) // per_block
        return c + (c % 2)

    ppi_chunks = _even_chunks(e_ppi)
    dti_chunks = _even_chunks(e_dti)
    ppi_total = ppi_chunks * per_block
    dti_total = dti_chunks * per_block
    dti_pad = dti_total - e_dti

    ps = _pad_idx(ppi_edge_index[0], ppi_total, 0, n_gene)
    pd = _pad_idx(ppi_edge_index[1], ppi_total, 0, n_gene)
    ds = _pad_idx(dti_src, dti_total, n_drug, _D)
    dd = _pad_idx(dti_dst, dti_total, 0, n_gene)

    edge_kernel = _make_edge_kernel(nw, ppi_chunks, dti_chunks)
    parts = edge_kernel(gene_n, drug_n, ps, pd, ds, dd)

    ppi_sum = jnp.sum(parts[:, 0, :])
    dti_sum = jnp.sum(parts[:, 1, :]) - jnp.float32(dti_pad)

    topology_loss = ppi_sum / e_ppi + dti_sum / e_dti
    dti_loss = bce_total / e_dti
    return dti_weight * dti_loss + topology_weight * topology_loss


# speedup vs baseline: 7.0033x; 1.0111x over previous
import dataclasses
import functools

import jax
import jax.numpy as jnp
from jax import lax
from jax.experimental import pallas as pl
from jax.experimental.pallas import tpu as pltpu
from jax.experimental.pallas import tpu_sc as plsc

_L = 16
_CH = 128
_D = 128


def _norm_rows(x):
    ss = jnp.sum(x * x, axis=1, keepdims=True)
    n = jnp.sqrt(ss)
    return (x / jnp.maximum(n, 1e-12)).astype(jnp.bfloat16)


def _dense_body(n_valid, drug_rows, gene_ref, drug_ref, p_ref, t_ref,
                gout_ref, dout_ref, bce_ref):
    gout_ref[...] = _norm_rows(gene_ref[...])
    nd = drug_ref.shape[0]
    dout_ref[:nd, :] = _norm_rows(drug_ref[...])
    dout_ref[nd:, :] = jnp.zeros((drug_rows - nd, drug_ref.shape[1]),
                                 jnp.bfloat16)
    p = p_ref[...]
    t = t_ref[...]
    term = jnp.maximum(p, 0.0) - p * t + jnp.log1p(jnp.exp(-jnp.abs(p)))
    rows, cols = p.shape
    idx = (lax.broadcasted_iota(jnp.int32, (rows, cols), 0) * cols
           + lax.broadcasted_iota(jnp.int32, (rows, cols), 1))
    term = jnp.where(idx < n_valid, term, 0.0)
    bce_ref[...] = jnp.sum(term, axis=0, keepdims=True)


def _dense_stage(gene_x, drug_x, p2d, t2d, n_valid, drug_rows):
    d = gene_x.shape[1]
    return pl.pallas_call(
        functools.partial(_dense_body, n_valid, drug_rows),
        out_shape=(
            jax.ShapeDtypeStruct(gene_x.shape, jnp.bfloat16),
            jax.ShapeDtypeStruct((drug_rows, d), jnp.bfloat16),
            jax.ShapeDtypeStruct((1, p2d.shape[1]), jnp.float32),
        ),
    )(gene_x, drug_x, p2d, t2d)


def _make_edge_kernel(nw, ppi_chunks, dti_chunks):
    ppw = ppi_chunks * _CH
    dtw = dti_chunks * _CH
    mesh = plsc.VectorSubcoreMesh(core_axis_name="c", subcore_axis_name="s")
    info = plsc.get_sparse_core_info()
    nc = info.num_cores

    cp = pltpu.CompilerParams()
    if "needs_layout_passes" in pltpu.CompilerParams.__dataclass_fields__:
        cp = dataclasses.replace(cp, needs_layout_passes=False)
    if "use_tc_tiling_on_sc" in pltpu.CompilerParams.__dataclass_fields__:
        cp = dataclasses.replace(cp, use_tc_tiling_on_sc=False)

    @functools.partial(
        pl.kernel,
        mesh=mesh,
        compiler_params=cp,
        out_type=jax.ShapeDtypeStruct((nw, 2, _L), jnp.float32),
        scratch_types=[
            pltpu.VMEM((ppw,), jnp.int32),
            pltpu.VMEM((ppw,), jnp.int32),
            pltpu.VMEM((dtw,), jnp.int32),
            pltpu.VMEM((dtw,), jnp.int32),
            pltpu.VMEM((_CH, _D // 2), jnp.float32),
            pltpu.VMEM((_CH, _D // 2), jnp.float32),
            pltpu.VMEM((_CH, _D // 2), jnp.float32),
            pltpu.VMEM((_CH, _D // 2), jnp.float32),
            pltpu.VMEM((2, _L), jnp.float32),
            pltpu.SemaphoreType.DMA,
            pltpu.SemaphoreType.DMA,
            pltpu.SemaphoreType.DMA,
            pltpu.SemaphoreType.DMA,
        ],
    )
    def edge_kernel(gene_hbm, drug_hbm, ps_hbm, pd_hbm, ds_hbm, dd_hbm,
                    out_hbm, psidx, pdidx, dsidx, ddidx,
                    srows_a, drows_a, srows_b, drows_b, ovec,
                    ss0, ss1, sd0, sd1):
        wid = lax.axis_index("s") * nc + lax.axis_index("c")

        pltpu.sync_copy(ps_hbm.at[pl.ds(wid * ppw, ppw)], psidx)
        pltpu.sync_copy(pd_hbm.at[pl.ds(wid * ppw, ppw)], pdidx)
        pltpu.sync_copy(ds_hbm.at[pl.ds(wid * dtw, dtw)], dsidx)
        pltpu.sync_copy(dd_hbm.at[pl.ds(wid * dtw, dtw)], ddidx)

        def compute(sb, db, acc):
            def edge(e, acc):
                prod = None
                for k in range(_D // 2 // _L):
                    s = plsc.bitcast(sb[e, pl.ds(k * _L, _L)], jnp.bfloat16)
                    t = plsc.bitcast(db[e, pl.ds(k * _L, _L)], jnp.bfloat16)
                    st = s * t
                    prod = st if prod is None else prod + st
                lo, hi = plsc.unpack(prod, format=plsc.PackFormat.INTERLEAVED)
                dt = jnp.sum(lo + hi)
                r = dt - 1.0
                return acc + r * r

            return lax.fori_loop(0, _CH, edge, acc)

        def run_class(tbl_s, tbl_d, sidx, didx, nch, acc):
            bufs = ((srows_a, drows_a, ss0, sd0),
                    (srows_b, drows_b, ss1, sd1))

            def start(t, b):
                sr, dr, sss, ssd = bufs[b]
                pltpu.async_copy(tbl_s.at[sidx.at[pl.ds(t * _CH, _CH)]],
                                 sr, sss)
                pltpu.async_copy(tbl_d.at[didx.at[pl.ds(t * _CH, _CH)]],
                                 dr, ssd)

            def wait(b):
                sr, dr, sss, ssd = bufs[b]
                pltpu.make_async_copy(tbl_s.at[sidx.at[pl.ds(0, _CH)]],
                                      sr, sss).wait()
                pltpu.make_async_copy(tbl_d.at[didx.at[pl.ds(0, _CH)]],
                                      dr, ssd).wait()

            start(0, 0)
            start(1, 1)

            def pair(i, acc):
                for b in (0, 1):
                    t = 2 * i + b
                    wait(b)

                    @pl.when(t + 2 < nch)
                    def _():
                        start(t + 2, b)

                    sr, dr = bufs[b][0], bufs[b][1]
                    acc = compute(sr, dr, acc)
                return acc

            return lax.fori_loop(0, nch // 2, pair, acc)

        acc_ppi = run_class(gene_hbm, gene_hbm, psidx, pdidx, ppi_chunks,
                            jnp.zeros((), jnp.float32))
        acc_dti = run_class(drug_hbm, gene_hbm, dsidx, ddidx, dti_chunks,
                            jnp.zeros((), jnp.float32))

        lane = lax.iota(jnp.int32, _L)
        ovec[0, :] = jnp.where(lane == 0, acc_ppi, 0.0)
        ovec[1, :] = jnp.where(lane == 0, acc_dti, 0.0)
        pltpu.sync_copy(ovec, out_hbm.at[wid])

    return edge_kernel


def _pad_idx(idx, total, fill_base, fill_mod):
    pad = total - idx.shape[0]
    if pad == 0:
        return idx.astype(jnp.int32)
    fill = fill_base + (jnp.arange(pad, dtype=jnp.int32) % fill_mod)
    return jnp.concatenate([idx.astype(jnp.int32), fill])


def kernel(gene_x, drug_x, predicted_dti, known_dti, ppi_edge_index,
           dti_src, dti_dst):
    dti_weight = 1.0
    topology_weight = 0.1

    n_gene, d = gene_x.shape
    n_drug = drug_x.shape[0]
    e_ppi = ppi_edge_index.shape[1]
    e_dti = predicted_dti.shape[0]

    info = plsc.get_sparse_core_info()
    nw = info.num_cores * info.num_subcores

    drug_rows = ((n_drug + _CH - 1) // _CH) * _CH + _CH
    cols = 128
    n_flat = ((e_dti + cols * 8 - 1) // (cols * 8)) * (cols * 8)
    p2d = jnp.pad(predicted_dti, (0, n_flat - e_dti)).reshape(-1, cols)
    t2d = jnp.pad(known_dti, (0, n_flat - e_dti)).reshape(-1, cols)
    gene_nb, drug_nb, bce_part = _dense_stage(gene_x, drug_x, p2d, t2d,
                                              e_dti, drug_rows)
    bce_total = jnp.sum(bce_part)

    def _pack(nrm):
        return lax.bitcast_convert_type(
            nrm.reshape(nrm.shape[0], d // 2, 2), jnp.float32)

    gene_n = _pack(gene_nb)
    drug_n = _pack(drug_nb)

    per_block = nw * _CH

    def _even_chunks(n):
        c = (n + per_block - 1) // per_block
        return c + (c % 2)

    ppi_chunks = _even_chunks(e_ppi)
    dti_chunks = _even_chunks(e_dti)
    ppi_total = ppi_chunks * per_block
    dti_total = dti_chunks * per_block
    dti_pad = dti_total - e_dti

    ps = _pad_idx(ppi_edge_index[0], ppi_total, 0, n_gene)
    pd = _pad_idx(ppi_edge_index[1], ppi_total, 0, n_gene)
    ds = _pad_idx(dti_src, dti_total, n_drug, _D)
    dd = _pad_idx(dti_dst, dti_total, 0, n_gene)

    edge_kernel = _make_edge_kernel(nw, ppi_chunks, dti_chunks)
    parts = edge_kernel(gene_n, drug_n, ps, pd, ds, dd)

    ppi_sum = jnp.sum(parts[:, 0, :])
    dti_sum = jnp.sum(parts[:, 1, :]) - jnp.float32(dti_pad)

    topology_loss = ppi_sum / e_ppi + dti_sum / e_dti
    dti_loss = bce_total / e_dti
    return dti_weight * dti_loss + topology_weight * topology_loss


# speedup vs baseline: 7.0186x; 1.0022x over previous
import dataclasses
import functools

import jax
import jax.numpy as jnp
from jax import lax
from jax.experimental import pallas as pl
from jax.experimental.pallas import tpu as pltpu
from jax.experimental.pallas import tpu_sc as plsc

_L = 16
_CH = 128
_D = 128


def _norm_rows(x):
    ss = jnp.sum(x * x, axis=1, keepdims=True)
    n = jnp.sqrt(ss)
    return (x / jnp.maximum(n, 1e-12)).astype(jnp.bfloat16)


def _dense_body(n_valid, drug_rows, gene_ref, drug_ref, p_ref, t_ref,
                gout_ref, dout_ref, bce_ref):
    gout_ref[...] = _norm_rows(gene_ref[...])
    nd = drug_ref.shape[0]
    dout_ref[:nd, :] = _norm_rows(drug_ref[...])
    dout_ref[nd:, :] = jnp.zeros((drug_rows - nd, drug_ref.shape[1]),
                                 jnp.bfloat16)
    p = p_ref[...]
    t = t_ref[...]
    term = jnp.maximum(p, 0.0) - p * t + jnp.log1p(jnp.exp(-jnp.abs(p)))
    rows, cols = p.shape
    idx = (lax.broadcasted_iota(jnp.int32, (rows, cols), 0) * cols
           + lax.broadcasted_iota(jnp.int32, (rows, cols), 1))
    term = jnp.where(idx < n_valid, term, 0.0)
    bce_ref[...] = jnp.sum(term, axis=0, keepdims=True)


def _dense_stage(gene_x, drug_x, p2d, t2d, n_valid, drug_rows):
    d = gene_x.shape[1]
    return pl.pallas_call(
        functools.partial(_dense_body, n_valid, drug_rows),
        out_shape=(
            jax.ShapeDtypeStruct(gene_x.shape, jnp.bfloat16),
            jax.ShapeDtypeStruct((drug_rows, d), jnp.bfloat16),
            jax.ShapeDtypeStruct((1, p2d.shape[1]), jnp.float32),
        ),
    )(gene_x, drug_x, p2d, t2d)


def _make_edge_kernel(nw, ppi_chunks, dti_chunks):
    ppw = ppi_chunks * _CH
    dtw = dti_chunks * _CH
    mesh = plsc.VectorSubcoreMesh(core_axis_name="c", subcore_axis_name="s")
    info = plsc.get_sparse_core_info()
    nc = info.num_cores

    cp = pltpu.CompilerParams()
    if "needs_layout_passes" in pltpu.CompilerParams.__dataclass_fields__:
        cp = dataclasses.replace(cp, needs_layout_passes=False)
    if "use_tc_tiling_on_sc" in pltpu.CompilerParams.__dataclass_fields__:
        cp = dataclasses.replace(cp, use_tc_tiling_on_sc=False)

    @functools.partial(
        pl.kernel,
        mesh=mesh,
        compiler_params=cp,
        out_type=jax.ShapeDtypeStruct((nw, 2, _L), jnp.float32),
        scratch_types=[
            pltpu.VMEM((ppw,), jnp.int32),
            pltpu.VMEM((ppw,), jnp.int32),
            pltpu.VMEM((dtw,), jnp.int32),
            pltpu.VMEM((dtw,), jnp.int32),
            pltpu.VMEM((_CH, _D // 2), jnp.float32),
            pltpu.VMEM((_CH, _D // 2), jnp.float32),
            pltpu.VMEM((_CH, _D // 2), jnp.float32),
            pltpu.VMEM((_CH, _D // 2), jnp.float32),
            pltpu.VMEM((2, _L), jnp.float32),
            pltpu.SemaphoreType.DMA,
            pltpu.SemaphoreType.DMA,
            pltpu.SemaphoreType.DMA,
            pltpu.SemaphoreType.DMA,
        ],
    )
    def edge_kernel(gene_hbm, drug_hbm, ps_hbm, pd_hbm, ds_hbm, dd_hbm,
                    out_hbm, psidx, pdidx, dsidx, ddidx,
                    srows_a, drows_a, srows_b, drows_b, ovec,
                    ss0, ss1, sd0, sd1):
        wid = lax.axis_index("s") * nc + lax.axis_index("c")

        pltpu.sync_copy(ps_hbm.at[pl.ds(wid * ppw, ppw)], psidx)
        pltpu.sync_copy(pd_hbm.at[pl.ds(wid * ppw, ppw)], pdidx)
        pltpu.sync_copy(ds_hbm.at[pl.ds(wid * dtw, dtw)], dsidx)
        pltpu.sync_copy(dd_hbm.at[pl.ds(wid * dtw, dtw)], ddidx)

        def compute(sb, db, acc):
            def one(e):
                prod = None
                for k in range(_D // 2 // _L):
                    s = plsc.bitcast(sb[e, pl.ds(k * _L, _L)], jnp.bfloat16)
                    t = plsc.bitcast(db[e, pl.ds(k * _L, _L)], jnp.bfloat16)
                    st = s * t
                    prod = st if prod is None else prod + st
                lo, hi = plsc.unpack(prod, format=plsc.PackFormat.INTERLEAVED)
                dt = jnp.sum(lo + hi)
                r = dt - 1.0
                return r * r

            def edge(e, acc):
                return acc + one(4 * e) + one(4 * e + 1) + one(4 * e + 2) + one(4 * e + 3)

            return lax.fori_loop(0, _CH // 4, edge, acc)

        def run_class(tbl_s, tbl_d, sidx, didx, nch, acc):
            bufs = ((srows_a, drows_a, ss0, sd0),
                    (srows_b, drows_b, ss1, sd1))

            def start(t, b):
                sr, dr, sss, ssd = bufs[b]
                pltpu.async_copy(tbl_s.at[sidx.at[pl.ds(t * _CH, _CH)]],
                                 sr, sss)
                pltpu.async_copy(tbl_d.at[didx.at[pl.ds(t * _CH, _CH)]],
                                 dr, ssd)

            def wait(b):
                sr, dr, sss, ssd = bufs[b]
                pltpu.make_async_copy(tbl_s.at[sidx.at[pl.ds(0, _CH)]],
                                      sr, sss).wait()
                pltpu.make_async_copy(tbl_d.at[didx.at[pl.ds(0, _CH)]],
                                      dr, ssd).wait()

            start(0, 0)
            start(1, 1)

            def pair(i, acc):
                for b in (0, 1):
                    t = 2 * i + b
                    wait(b)

                    @pl.when(t + 2 < nch)
                    def _():
                        start(t + 2, b)

                    sr, dr = bufs[b][0], bufs[b][1]
                    acc = compute(sr, dr, acc)
                return acc

            return lax.fori_loop(0, nch // 2, pair, acc)

        acc_ppi = run_class(gene_hbm, gene_hbm, psidx, pdidx, ppi_chunks,
                            jnp.zeros((), jnp.float32))
        acc_dti = run_class(drug_hbm, gene_hbm, dsidx, ddidx, dti_chunks,
                            jnp.zeros((), jnp.float32))

        lane = lax.iota(jnp.int32, _L)
        ovec[0, :] = jnp.where(lane == 0, acc_ppi, 0.0)
        ovec[1, :] = jnp.where(lane == 0, acc_dti, 0.0)
        pltpu.sync_copy(ovec, out_hbm.at[wid])

    return edge_kernel


def _pad_idx(idx, total, fill_base, fill_mod):
    pad = total - idx.shape[0]
    if pad == 0:
        return idx.astype(jnp.int32)
    fill = fill_base + (jnp.arange(pad, dtype=jnp.int32) % fill_mod)
    return jnp.concatenate([idx.astype(jnp.int32), fill])


def kernel(gene_x, drug_x, predicted_dti, known_dti, ppi_edge_index,
           dti_src, dti_dst):
    dti_weight = 1.0
    topology_weight = 0.1

    n_gene, d = gene_x.shape
    n_drug = drug_x.shape[0]
    e_ppi = ppi_edge_index.shape[1]
    e_dti = predicted_dti.shape[0]

    info = plsc.get_sparse_core_info()
    nw = info.num_cores * info.num_subcores

    drug_rows = ((n_drug + _CH - 1) // _CH) * _CH + _CH
    cols = 128
    n_flat = ((e_dti + cols * 8 - 1) // (cols * 8)) * (cols * 8)
    p2d = jnp.pad(predicted_dti, (0, n_flat - e_dti)).reshape(-1, cols)
    t2d = jnp.pad(known_dti, (0, n_flat - e_dti)).reshape(-1, cols)
    gene_nb, drug_nb, bce_part = _dense_stage(gene_x, drug_x, p2d, t2d,
                                              e_dti, drug_rows)
    bce_total = jnp.sum(bce_part)

    def _pack(nrm):
        return lax.bitcast_convert_type(
            nrm.reshape(nrm.shape[0], d // 2, 2), jnp.float32)

    gene_n = _pack(gene_nb)
    drug_n = _pack(drug_nb)

    per_block = nw * _CH

    def _even_chunks(n):
        c = (n + per_block - 1) // per_block
        return c + (c % 2)

    ppi_chunks = _even_chunks(e_ppi)
    dti_chunks = _even_chunks(e_dti)
    ppi_total = ppi_chunks * per_block
    dti_total = dti_chunks * per_block
    dti_pad = dti_total - e_dti

    ps = _pad_idx(ppi_edge_index[0], ppi_total, 0, n_gene)
    pd = _pad_idx(ppi_edge_index[1], ppi_total, 0, n_gene)
    ds = _pad_idx(dti_src, dti_total, n_drug, _D)
    dd = _pad_idx(dti_dst, dti_total, 0, n_gene)

    edge_kernel = _make_edge_kernel(nw, ppi_chunks, dti_chunks)
    parts = edge_kernel(gene_n, drug_n, ps, pd, ds, dd)

    ppi_sum = jnp.sum(parts[:, 0, :])
    dti_sum = jnp.sum(parts[:, 1, :]) - jnp.float32(dti_pad)

    topology_loss = ppi_sum / e_ppi + dti_sum / e_dti
    dti_loss = bce_total / e_dti
    return dti_weight * dti_loss + topology_weight * topology_loss
